# baseline (speedup 1.0000x reference)
import functools

import numpy as np

import jax
import jax.numpy as jnp
from jax import lax
from jax.experimental import pallas as pl
from jax.experimental.pallas import tpu as pltpu
from jax.experimental.pallas import tpu_sc as plsc

_T, _P, _C, _PS = 2, 65536, 3, 7
_HP, _WP = 256, 256
_N = _T * _P
_NPIX = _HP * _WP
_WN = _T * _NPIX
_PP = _PS * _PS
_PATCH = _C * _PP

_NC, _NS = 2, 16
_NWORK = _NC * _NS
_PPW = _N // _NWORK
_CHUNK = 128
_NCHUNK = _PPW // _CHUNK
_CW = _CHUNK * _PP
_CV = _CHUNK * _PATCH

_K3 = 30
_RSTRIPE = _WN // _NS
_ZROWS = 512

_mesh = plsc.VectorSubcoreMesh(core_axis_name="c", subcore_axis_name="s")

_VSTARTS = (0, 16, 32, 48, 64, 80, 96, 112, 128, 131)
_WSTARTS = (0, 16, 32, 33)

_OFFC_TAB = np.concatenate([
    np.array([(dy * _WP + dx) * 4 + 3
              for dy in range(_PS) for dx in range(_PS)],
             np.int32)[s:s + 16]
    for s in _WSTARTS])
_OFFV_TAB = np.concatenate([
    np.array([(dy * _WP + dx) * 4 + c
              for c in range(_C)
              for dy in range(_PS) for dx in range(_PS)],
             np.int32)[s:s + 16]
    for s in _VSTARTS])


@functools.partial(
    pl.kernel,
    out_type=jax.ShapeDtypeStruct((_NC * _WN * 4,), jnp.float32),
    mesh=_mesh,
    scratch_types=[
        pltpu.VMEM_SHARED((_WN * 4,), jnp.float32),
        pltpu.VMEM((_CV,), jnp.float32),
        pltpu.VMEM((_CV,), jnp.int32),
        pltpu.VMEM((_CV,), jnp.float32),
        pltpu.VMEM((_CV,), jnp.int32),
        pltpu.VMEM((_CW,), jnp.float32),
        pltpu.VMEM((_CW,), jnp.int32),
        pltpu.VMEM((_CW,), jnp.int32),
        pltpu.SemaphoreType.DMA,
        pltpu.SemaphoreType.DMA,
        pltpu.VMEM((_CHUNK,), jnp.int32),
        pltpu.VMEM((_CHUNK,), jnp.int32),
        pltpu.VMEM((_CHUNK,), jnp.int32),
        pltpu.VMEM((_CHUNK,), jnp.int32),
        pltpu.VMEM((_CHUNK,), jnp.int32),
        pltpu.VMEM((len(_VSTARTS) * 16,), jnp.int32),
        pltpu.VMEM((len(_WSTARTS) * 16,), jnp.int32),
        pltpu.VMEM((2048,), jnp.float32),
    ],
)
def _fold(xf, nli, offv_h, offc_h, zeros_h, acc_out, acc_sh,
          xbufa, idxva, xbufb, idxvb, ones, idxca, idxcb, sema, semb,
          tbuf, hbuf, wbuf, idx3, basesv, offv, offc, zbuf):
    core = lax.axis_index("c")
    sub = lax.axis_index("s")
    wid = core * _NS + sub
    one16 = jnp.ones((16,), jnp.float32)

    pltpu.sync_copy(zeros_h, zbuf)
    pltpu.sync_copy(offv_h, offv)
    pltpu.sync_copy(offc_h, offc)
    g0 = sub * (_RSTRIPE * 4)

    def ob(i, carry):
        ones[pl.ds(i * 16, 16)] = one16
        return carry
    lax.fori_loop(0, _CW // 16, ob, 0)

    def zv(i, carry):
        pltpu.sync_copy(zbuf, acc_sh.at[pl.ds(g0 + i * 2048, 2048)])
        return carry
    lax.fori_loop(0, (_RSTRIPE * 4) // 2048, zv, 0)

    plsc.subcore_barrier()

    lanes30 = lax.iota(jnp.int32, 16) * 30

    def load_gen(c, xbuf, idxv, idxc):
        pbase = wid * _PPW + c * _CHUNK

        def gi(k, c2):
            base30 = (pbase + k * 16) * 30
            idx3[pl.ds(k * 16, 16)] = lanes30 + base30
            return c2
        lax.fori_loop(0, _CHUNK // 16, gi, 0)
        pltpu.sync_copy(nli.at[idx3], tbuf)

        def gi1(k, c2):
            base30 = (pbase + k * 16) * 30 + 1
            idx3[pl.ds(k * 16, 16)] = lanes30 + base30
            return c2
        lax.fori_loop(0, _CHUNK // 16, gi1, 0)
        pltpu.sync_copy(nli.at[idx3], hbuf)

        def gi2(k, c2):
            base30 = (pbase + k * 16) * 30 + 2
            idx3[pl.ds(k * 16, 16)] = lanes30 + base30
            return c2
        lax.fori_loop(0, _CHUNK // 16, gi2, 0)
        pltpu.sync_copy(nli.at[idx3], wbuf)
        pltpu.sync_copy(xf.at[pl.ds(pbase * _PATCH, _CV)], xbuf)

        def pb(e, c2):
            sl = pl.ds(e * 16, 16)
            basesv[sl] = (tbuf[sl] * _NPIX + hbuf[sl] * _WP + wbuf[sl]) * 4
            return c2
        lax.fori_loop(0, _CHUNK // 16, pb, 0)

        def pp(e, c2):
            bvv = basesv[pl.ds(e * 16, 16)]
            for lane in range(16):
                bv = bvv[lane]
                s0 = (e * 16 + lane) * _PATCH
                for si, s in enumerate(_VSTARTS):
                    idxv[pl.ds(s0 + s, 16)] = offv[pl.ds(si * 16, 16)] + bv
                sw = (e * 16 + lane) * _PP
                for si, s in enumerate(_WSTARTS):
                    idxc[pl.ds(sw + s, 16)] = offc[pl.ds(si * 16, 16)] + bv
            return c2
        lax.fori_loop(0, _CHUNK // 16, pp, 0)

    def chunk_body(i, carry):
        load_gen(i * 2, xbufa, idxva, idxca)
        da1 = pltpu.async_copy(xbufa, acc_sh.at[idxva], sema, add=True)
        da2 = pltpu.async_copy(ones, acc_sh.at[idxca], sema, add=True)
        load_gen(i * 2 + 1, xbufb, idxvb, idxcb)
        da1.wait()
        da2.wait()
        db1 = pltpu.async_copy(xbufb, acc_sh.at[idxvb], semb, add=True)
        db2 = pltpu.async_copy(ones, acc_sh.at[idxcb], semb, add=True)
        db1.wait()
        db2.wait()
        return carry
    lax.fori_loop(0, _NCHUNK // 2, chunk_body, 0)

    plsc.subcore_barrier()
    pltpu.sync_copy(acc_sh.at[pl.ds(g0, _RSTRIPE * 4)],
                    acc_out.at[pl.ds(core * _WN * 4 + g0, _RSTRIPE * 4)])


def _norm_body(p0_ref, p1_ref, o_ref):
    v = p0_ref[...] + p1_ref[...]
    lane = lax.broadcasted_iota(jnp.int32, v.shape, 1) % 4
    r1 = jnp.roll(v, -1, axis=1)
    r2 = jnp.roll(v, -2, axis=1)
    r3 = jnp.roll(v, -3, axis=1)
    cnt = jnp.where(lane == 3, v,
                    jnp.where(lane == 2, r1,
                              jnp.where(lane == 1, r2, r3)))
    o_ref[...] = v / jnp.maximum(cnt, 1e-10)


_norm = pl.pallas_call(
    _norm_body,
    out_shape=jax.ShapeDtypeStruct((_WN * 4 // 128, 128), jnp.float32),
)


@functools.partial(
    pl.kernel,
    out_type=jax.ShapeDtypeStruct((_N * _PATCH,), jnp.float32),
    mesh=_mesh,
    scratch_types=[
        pltpu.VMEM_SHARED((_WN * 4,), jnp.float32),
        pltpu.VMEM((_CV,), jnp.float32),
        pltpu.VMEM((_CV,), jnp.int32),
        pltpu.VMEM((_CV,), jnp.float32),
        pltpu.VMEM((_CV,), jnp.int32),
        pltpu.SemaphoreType.DMA,
        pltpu.SemaphoreType.DMA,
        pltpu.VMEM((_CHUNK,), jnp.int32),
        pltpu.VMEM((_CHUNK,), jnp.int32),
        pltpu.VMEM((_CHUNK,), jnp.int32),
        pltpu.VMEM((_CHUNK,), jnp.int32),
        pltpu.VMEM((_CHUNK,), jnp.int32),
        pltpu.VMEM((len(_VSTARTS) * 16,), jnp.int32),
    ],
)
def _unfold(img, nli, offv_h, out, img_sh,
            obufa, idxva, obufb, idxvb, sema, semb,
            tbuf, hbuf, wbuf, idx3, basesv, offv):
    core = lax.axis_index("c")
    sub = lax.axis_index("s")
    wid = core * _NS + sub

    pltpu.sync_copy(offv_h, offv)
    g0 = sub * (_RSTRIPE * 4)
    pltpu.sync_copy(img.at[pl.ds(g0, _RSTRIPE * 4)],
                    img_sh.at[pl.ds(g0, _RSTRIPE * 4)])
    plsc.subcore_barrier()

    lanes30 = lax.iota(jnp.int32, 16) * 30

    def gen(c, idxv):
        pbase = wid * _PPW + c * _CHUNK

        def gi(k, c2):
            base30 = (pbase + k * 16) * 30
            idx3[pl.ds(k * 16, 16)] = lanes30 + base30
            return c2
        lax.fori_loop(0, _CHUNK // 16, gi, 0)
        pltpu.sync_copy(nli.at[idx3], tbuf)

        def gi1(k, c2):
            base30 = (pbase + k * 16) * 30 + 1
            idx3[pl.ds(k * 16, 16)] = lanes30 + base30
            return c2
        lax.fori_loop(0, _CHUNK // 16, gi1, 0)
        pltpu.sync_copy(nli.at[idx3], hbuf)

        def gi2(k, c2):
            base30 = (pbase + k * 16) * 30 + 2
            idx3[pl.ds(k * 16, 16)] = lanes30 + base30
            return c2
        lax.fori_loop(0, _CHUNK // 16, gi2, 0)
        pltpu.sync_copy(nli.at[idx3], wbuf)

        def pb(e, c2):
            sl = pl.ds(e * 16, 16)
            basesv[sl] = (tbuf[sl] * _NPIX + hbuf[sl] * _WP + wbuf[sl]) * 4
            return c2
        lax.fori_loop(0, _CHUNK // 16, pb, 0)

        def pp(e, c2):
            bvv = basesv[pl.ds(e * 16, 16)]
            for lane in range(16):
                bv = bvv[lane]
                s0 = (e * 16 + lane) * _PATCH
                for si, s in enumerate(_VSTARTS):
                    idxv[pl.ds(s0 + s, 16)] = offv[pl.ds(si * 16, 16)] + bv
            return c2
        lax.fori_loop(0, _CHUNK // 16, pp, 0)
        return pbase

    def chunk_body(i, carry):
        pba = gen(i * 2, idxva)
        ga = pltpu.async_copy(img_sh.at[idxva], obufa, sema)
        pbb = gen(i * 2 + 1, idxvb)
        ga.wait()
        oa = pltpu.async_copy(obufa, out.at[pl.ds(pba * _PATCH, _CV)], sema)
        gb = pltpu.async_copy(img_sh.at[idxvb], obufb, semb)
        oa.wait()
        gb.wait()
        ob = pltpu.async_copy(obufb, out.at[pl.ds(pbb * _PATCH, _CV)], semb)
        ob.wait()
        return carry
    lax.fori_loop(0, _NCHUNK // 2, chunk_body, 0)


def kernel(x, nlDists, nlInds, pixels_h, pixels_w, both):
    xf = x.reshape(_N * _PATCH)
    nli = nlInds.astype(jnp.int32).reshape(_N * _K3)
    offv_h = jnp.asarray(_OFFV_TAB)
    offc_h = jnp.asarray(_OFFC_TAB)
    zeros_h = jnp.zeros((2048,), jnp.float32)

    acc = _fold(xf, nli, offv_h, offc_h, zeros_h)
    acc2 = acc.reshape(_NC, _WN * 4 // 128, 128)
    img = _norm(acc2[0], acc2[1])
    outf = _unfold(img.reshape(_WN * 4), nli, offv_h)
    return outf.reshape(_T, _P, 1, _PATCH)


# speedup vs baseline: 4.2850x; 4.2850x over previous
import functools

import numpy as np

import jax
import jax.numpy as jnp
from jax import lax
from jax.experimental import pallas as pl
from jax.experimental.pallas import tpu as pltpu
from jax.experimental.pallas import tpu_sc as plsc

_T, _P, _C, _PS = 2, 65536, 3, 7
_HP, _WP = 256, 256
_N = _T * _P
_NPIX = _HP * _WP
_WN = _T * _NPIX
_PP = _PS * _PS
_PATCH = _C * _PP

_NC, _NS = 2, 16
_NWORK = _NC * _NS
_PPW = _N // _NWORK
_CHUNK = 128
_NCHUNK = _PPW // _CHUNK
_CW = _CHUNK * _PP
_CV = _CHUNK * _PATCH

_RSTRIPE = _WN // _NS
_ZROWS = 512

_mesh = plsc.VectorSubcoreMesh(core_axis_name="c", subcore_axis_name="s")

_VSTARTS = (0, 16, 32, 48, 64, 80, 96, 112, 128, 131)
_WSTARTS = (0, 16, 32, 33)

_OFFC_TAB = np.concatenate([
    np.array([(dy * _WP + dx) * 4 + 3
              for dy in range(_PS) for dx in range(_PS)],
             np.int32)[s:s + 16]
    for s in _WSTARTS])
_OFFV_TAB = np.concatenate([
    np.array([(dy * _WP + dx) * 4 + c
              for c in range(_C)
              for dy in range(_PS) for dx in range(_PS)],
             np.int32)[s:s + 16]
    for s in _VSTARTS])


@functools.partial(
    pl.kernel,
    out_type=jax.ShapeDtypeStruct((_NC * _WN * 4,), jnp.float32),
    mesh=_mesh,
    scratch_types=[
        pltpu.VMEM_SHARED((_WN * 4,), jnp.float32),
        pltpu.VMEM((_CV,), jnp.float32),
        pltpu.VMEM((_CV,), jnp.int32),
        pltpu.VMEM((_CV,), jnp.float32),
        pltpu.VMEM((_CV,), jnp.int32),
        pltpu.VMEM((_CW,), jnp.float32),
        pltpu.VMEM((_CW,), jnp.int32),
        pltpu.VMEM((_CW,), jnp.int32),
        pltpu.SemaphoreType.DMA,
        pltpu.SemaphoreType.DMA,
        pltpu.VMEM((_CHUNK,), jnp.int32),
        pltpu.VMEM((_CHUNK,), jnp.int32),
        pltpu.VMEM((_CHUNK,), jnp.int32),
        pltpu.VMEM((_CHUNK,), jnp.int32),
        pltpu.VMEM((len(_VSTARTS) * 16,), jnp.int32),
        pltpu.VMEM((len(_WSTARTS) * 16,), jnp.int32),
        pltpu.VMEM((2048,), jnp.float32),
    ],
)
def _fold(xf, tiv, hiv, wiv, offv_h, offc_h, zeros_h, acc_out, acc_sh,
          xbufa, idxva, xbufb, idxvb, ones, idxca, idxcb, sema, semb,
          tbuf, hbuf, wbuf, basesv, offv, offc, zbuf):
    core = lax.axis_index("c")
    sub = lax.axis_index("s")
    wid = core * _NS + sub
    one16 = jnp.ones((16,), jnp.float32)

    pltpu.sync_copy(zeros_h, zbuf)
    pltpu.sync_copy(offv_h, offv)
    pltpu.sync_copy(offc_h, offc)
    g0 = sub * (_RSTRIPE * 4)

    def ob(i, carry):
        ones[pl.ds(i * 16, 16)] = one16
        return carry
    lax.fori_loop(0, _CW // 16, ob, 0)

    def zv(i, carry):
        pltpu.sync_copy(zbuf, acc_sh.at[pl.ds(g0 + i * 2048, 2048)])
        return carry
    lax.fori_loop(0, (_RSTRIPE * 4) // 2048, zv, 0)

    plsc.subcore_barrier()

    def load_gen(c, xbuf, idxv, idxc):
        pbase = wid * _PPW + c * _CHUNK
        pltpu.sync_copy(tiv.at[pl.ds(pbase, _CHUNK)], tbuf)
        pltpu.sync_copy(hiv.at[pl.ds(pbase, _CHUNK)], hbuf)
        pltpu.sync_copy(wiv.at[pl.ds(pbase, _CHUNK)], wbuf)
        pltpu.sync_copy(xf.at[pl.ds(pbase * _PATCH, _CV)], xbuf)

        def pb(e, c2):
            sl = pl.ds(e * 16, 16)
            basesv[sl] = (tbuf[sl] * _NPIX + hbuf[sl] * _WP + wbuf[sl]) * 4
            return c2
        lax.fori_loop(0, _CHUNK // 16, pb, 0)

        def pp(e, c2):
            bvv = basesv[pl.ds(e * 16, 16)]
            for lane in range(16):
                bv = bvv[lane]
                s0 = (e * 16 + lane) * _PATCH
                for si, s in enumerate(_VSTARTS):
                    idxv[pl.ds(s0 + s, 16)] = offv[pl.ds(si * 16, 16)] + bv
                sw = (e * 16 + lane) * _PP
                for si, s in enumerate(_WSTARTS):
                    idxc[pl.ds(sw + s, 16)] = offc[pl.ds(si * 16, 16)] + bv
            return c2
        lax.fori_loop(0, _CHUNK // 16, pp, 0)

    def chunk_body(i, carry):
        load_gen(i * 2, xbufa, idxva, idxca)
        da1 = pltpu.async_copy(xbufa, acc_sh.at[idxva], sema, add=True)
        da2 = pltpu.async_copy(ones, acc_sh.at[idxca], sema, add=True)
        load_gen(i * 2 + 1, xbufb, idxvb, idxcb)
        da1.wait()
        da2.wait()
        db1 = pltpu.async_copy(xbufb, acc_sh.at[idxvb], semb, add=True)
        db2 = pltpu.async_copy(ones, acc_sh.at[idxcb], semb, add=True)
        db1.wait()
        db2.wait()
        return carry
    lax.fori_loop(0, _NCHUNK // 2, chunk_body, 0)

    plsc.subcore_barrier()
    pltpu.sync_copy(acc_sh.at[pl.ds(g0, _RSTRIPE * 4)],
                    acc_out.at[pl.ds(core * _WN * 4 + g0, _RSTRIPE * 4)])


def _norm_body(p0_ref, p1_ref, o_ref):
    v = p0_ref[...] + p1_ref[...]
    lane = lax.broadcasted_iota(jnp.int32, v.shape, 1) % 4
    r1 = jnp.roll(v, -1, axis=1)
    r2 = jnp.roll(v, -2, axis=1)
    r3 = jnp.roll(v, -3, axis=1)
    cnt = jnp.where(lane == 3, v,
                    jnp.where(lane == 2, r1,
                              jnp.where(lane == 1, r2, r3)))
    o_ref[...] = v / jnp.maximum(cnt, 1e-10)


_norm = pl.pallas_call(
    _norm_body,
    out_shape=jax.ShapeDtypeStruct((_WN * 4 // 128, 128), jnp.float32),
)


@functools.partial(
    pl.kernel,
    out_type=jax.ShapeDtypeStruct((_N * _PATCH,), jnp.float32),
    mesh=_mesh,
    scratch_types=[
        pltpu.VMEM_SHARED((_WN * 4,), jnp.float32),
        pltpu.VMEM((_CV,), jnp.float32),
        pltpu.VMEM((_CV,), jnp.int32),
        pltpu.VMEM((_CV,), jnp.float32),
        pltpu.VMEM((_CV,), jnp.int32),
        pltpu.SemaphoreType.DMA,
        pltpu.SemaphoreType.DMA,
        pltpu.VMEM((_CHUNK,), jnp.int32),
        pltpu.VMEM((_CHUNK,), jnp.int32),
        pltpu.VMEM((_CHUNK,), jnp.int32),
        pltpu.VMEM((_CHUNK,), jnp.int32),
        pltpu.VMEM((len(_VSTARTS) * 16,), jnp.int32),
    ],
)
def _unfold(img, tiv, hiv, wiv, offv_h, out, img_sh,
            obufa, idxva, obufb, idxvb, sema, semb,
            tbuf, hbuf, wbuf, basesv, offv):
    core = lax.axis_index("c")
    sub = lax.axis_index("s")
    wid = core * _NS + sub

    pltpu.sync_copy(offv_h, offv)
    g0 = sub * (_RSTRIPE * 4)
    pltpu.sync_copy(img.at[pl.ds(g0, _RSTRIPE * 4)],
                    img_sh.at[pl.ds(g0, _RSTRIPE * 4)])
    plsc.subcore_barrier()

    def gen(c, idxv):
        pbase = wid * _PPW + c * _CHUNK
        pltpu.sync_copy(tiv.at[pl.ds(pbase, _CHUNK)], tbuf)
        pltpu.sync_copy(hiv.at[pl.ds(pbase, _CHUNK)], hbuf)
        pltpu.sync_copy(wiv.at[pl.ds(pbase, _CHUNK)], wbuf)

        def pb(e, c2):
            sl = pl.ds(e * 16, 16)
            basesv[sl] = (tbuf[sl] * _NPIX + hbuf[sl] * _WP + wbuf[sl]) * 4
            return c2
        lax.fori_loop(0, _CHUNK // 16, pb, 0)

        def pp(e, c2):
            bvv = basesv[pl.ds(e * 16, 16)]
            for lane in range(16):
                bv = bvv[lane]
                s0 = (e * 16 + lane) * _PATCH
                for si, s in enumerate(_VSTARTS):
                    idxv[pl.ds(s0 + s, 16)] = offv[pl.ds(si * 16, 16)] + bv
            return c2
        lax.fori_loop(0, _CHUNK // 16, pp, 0)
        return pbase

    def chunk_body(i, carry):
        pba = gen(i * 2, idxva)
        ga = pltpu.async_copy(img_sh.at[idxva], obufa, sema)
        pbb = gen(i * 2 + 1, idxvb)
        ga.wait()
        oa = pltpu.async_copy(obufa, out.at[pl.ds(pba * _PATCH, _CV)], sema)
        gb = pltpu.async_copy(img_sh.at[idxvb], obufb, semb)
        oa.wait()
        gb.wait()
        ob = pltpu.async_copy(obufb, out.at[pl.ds(pbb * _PATCH, _CV)], semb)
        ob.wait()
        return carry
    lax.fori_loop(0, _NCHUNK // 2, chunk_body, 0)


def kernel(x, nlDists, nlInds, pixels_h, pixels_w, both):
    xf = x.reshape(_N * _PATCH)
    inds = nlInds[:, :, 0, :].reshape(_N, 3).astype(jnp.int32)
    tiv = inds[:, 0]
    hiv = inds[:, 1]
    wiv = inds[:, 2]
    offv_h = jnp.asarray(_OFFV_TAB)
    offc_h = jnp.asarray(_OFFC_TAB)
    zeros_h = jnp.zeros((2048,), jnp.float32)

    acc = _fold(xf, tiv, hiv, wiv, offv_h, offc_h, zeros_h)
    acc2 = acc.reshape(_NC, _WN * 4 // 128, 128)
    img = _norm(acc2[0], acc2[1])
    outf = _unfold(img.reshape(_WN * 4), tiv, hiv, wiv, offv_h)
    return outf.reshape(_T, _P, 1, _PATCH)
